# baseline (speedup 1.0000x reference)
import functools

import jax
import jax.numpy as jnp
from jax import lax
from jax.experimental import pallas as pl
from jax.experimental.pallas import tpu as pltpu
from jax.experimental.pallas import tpu_sc as plsc

NUM_CLASSES = 10000
CLASSES_PER_TASK = 1000
CURRENT_TASK = 1
BAND_START = CURRENT_TASK * CLASSES_PER_TASK
BAND_END = BAND_START + CLASSES_PER_TASK

LANES = 16
HULL0 = (BAND_START // LANES) * LANES
N_FULL = (BAND_END - (HULL0 + LANES)) // LANES

ROWS = 4096
N_WORKERS = 32
ROWS_PER_WORKER = ROWS // N_WORKERS
ROW_CHUNK = 1
N_CHUNKS = ROWS_PER_WORKER // ROW_CHUNK
SLOTS = 8
PRIME = 4


def _sc_body(x_hbm, alpha_hbm, beta_hbm, o_hbm, buf, ab_v, in_sem, out_sem):
    wid = lax.axis_index("s") * 2 + lax.axis_index("c")
    base = wid * ROWS_PER_WORKER

    def in_dma(k, slot):
        return pltpu.make_async_copy(
            x_hbm.at[pl.ds(base + k * ROW_CHUNK, ROW_CHUNK), :],
            buf.at[slot],
            in_sem.at[slot],
        )

    def out_dma(k, slot):
        return pltpu.make_async_copy(
            buf.at[slot],
            o_hbm.at[pl.ds(base + k * ROW_CHUNK, ROW_CHUNK), :],
            out_sem.at[slot],
        )

    pltpu.sync_copy(alpha_hbm, ab_v.at[pl.ds(0, 1)])
    pltpu.sync_copy(beta_hbm, ab_v.at[pl.ds(8, 1)])
    ab = ab_v[...]
    a = ab[0]
    b = ab[8]
    edge_mask = lax.iota(jnp.int32, LANES) >= (BAND_START - HULL0)

    for d in range(PRIME):
        in_dma(d, d).start()

    def correct_rows(slot):
        for r in range(ROW_CHUNK):
            v = buf[slot, r, pl.ds(HULL0, LANES)]
            buf[slot, r, pl.ds(HULL0, LANES)] = jnp.where(
                edge_mask, v * a + b, v)
            for j in range(N_FULL):
                off = (HULL0 + LANES) + j * LANES
                buf[slot, r, pl.ds(off, LANES)] = (
                    buf[slot, r, pl.ds(off, LANES)] * a + b)

    def outer(k0, _):
        for d in range(SLOTS):
            k = k0 + d

            @pl.when(k + PRIME < N_CHUNKS)
            def _():
                @pl.when(k + PRIME >= SLOTS)
                def _():
                    out_dma(k + PRIME - SLOTS, (k + PRIME) % SLOTS).wait()

                in_dma(k + PRIME, (k + PRIME) % SLOTS).start()

            in_dma(k, d).wait()
            correct_rows(d)
            out_dma(k, d).start()
        return 0

    lax.fori_loop(0, N_CHUNKS // SLOTS, lambda i, c: outer(i * SLOTS, c), 0)

    for k in range(N_CHUNKS - SLOTS, N_CHUNKS):
        out_dma(k, k % SLOTS).wait()


def kernel(x, alpha, beta):
    m, n = x.shape
    mesh = plsc.VectorSubcoreMesh(core_axis_name="c", subcore_axis_name="s")
    sc_kernel = functools.partial(
        pl.kernel,
        mesh=mesh,
        out_type=jax.ShapeDtypeStruct((m, n), x.dtype),
        scratch_types=[
            pltpu.VMEM((SLOTS, ROW_CHUNK, NUM_CLASSES), jnp.float32),
            pltpu.VMEM((16,), jnp.float32),
            pltpu.SemaphoreType.DMA((SLOTS,)),
            pltpu.SemaphoreType.DMA((SLOTS,)),
        ],
    )(_sc_body)
    return sc_kernel(x, alpha, beta)


# speedup vs baseline: 1.1890x; 1.1890x over previous
import functools

import jax
import jax.numpy as jnp
from jax import lax
from jax.experimental import pallas as pl
from jax.experimental.pallas import tpu as pltpu
from jax.experimental.pallas import tpu_sc as plsc

NUM_CLASSES = 10000
CLASSES_PER_TASK = 1000
CURRENT_TASK = 1
BAND_START = CURRENT_TASK * CLASSES_PER_TASK
BAND_END = BAND_START + CLASSES_PER_TASK

LANES = 16
HULL_START = (BAND_START // 128) * 128
HULL_END = -(-BAND_END // 128) * 128
HULL_W = HULL_END - HULL_START
LOC0 = ((BAND_START - HULL_START) // LANES) * LANES
N_FULL = (BAND_END - HULL_START - (LOC0 + LANES)) // LANES

ROWS = 4096
N_WORKERS = 32
ROWS_PER_WORKER = ROWS // N_WORKERS
ROW_CHUNK = 8
N_CHUNKS = ROWS_PER_WORKER // ROW_CHUNK
SLOTS = 4
PRIME = 2


def _sc_body(x_hbm, alpha_hbm, beta_hbm, c_hbm, buf, ab_v, in_sem, out_sem):
    wid = lax.axis_index("s") * 2 + lax.axis_index("c")
    base = wid * ROWS_PER_WORKER

    def in_dma(k, slot):
        return pltpu.make_async_copy(
            x_hbm.at[pl.ds(base + k * ROW_CHUNK, ROW_CHUNK),
                     pl.ds(HULL_START, HULL_W)],
            buf.at[slot],
            in_sem.at[slot],
        )

    def out_dma(k, slot):
        return pltpu.make_async_copy(
            buf.at[slot],
            c_hbm.at[pl.ds(base + k * ROW_CHUNK, ROW_CHUNK), :],
            out_sem.at[slot],
        )

    pltpu.sync_copy(alpha_hbm, ab_v.at[pl.ds(0, 1)])
    pltpu.sync_copy(beta_hbm, ab_v.at[pl.ds(8, 1)])
    ab = ab_v[...]
    a = ab[0]
    b = ab[8]
    edge_mask = lax.iota(jnp.int32, LANES) >= (BAND_START - HULL_START - LOC0)

    for d in range(PRIME):
        in_dma(d, d).start()

    def correct_rows(slot):
        for r in range(ROW_CHUNK):
            v = buf[slot, r, pl.ds(LOC0, LANES)]
            buf[slot, r, pl.ds(LOC0, LANES)] = jnp.where(
                edge_mask, v * a + b, v)
            for j in range(N_FULL):
                off = (LOC0 + LANES) + j * LANES
                buf[slot, r, pl.ds(off, LANES)] = (
                    buf[slot, r, pl.ds(off, LANES)] * a + b)

    def outer(k0, _):
        for d in range(SLOTS):
            k = k0 + d

            @pl.when(k + PRIME < N_CHUNKS)
            def _():
                @pl.when(k + PRIME >= SLOTS)
                def _():
                    out_dma(k + PRIME - SLOTS, (k + PRIME) % SLOTS).wait()

                in_dma(k + PRIME, (k + PRIME) % SLOTS).start()

            in_dma(k, d).wait()
            correct_rows(d)
            out_dma(k, d).start()
        return 0

    lax.fori_loop(0, N_CHUNKS // SLOTS, lambda i, c: outer(i * SLOTS, c), 0)

    for k in range(N_CHUNKS - SLOTS, N_CHUNKS):
        out_dma(k, k % SLOTS).wait()


def _sc_band_hull(x, alpha, beta):
    mesh = plsc.VectorSubcoreMesh(core_axis_name="c", subcore_axis_name="s")
    sc_kernel = functools.partial(
        pl.kernel,
        mesh=mesh,
        out_type=jax.ShapeDtypeStruct((ROWS, HULL_W), jnp.float32),
        scratch_types=[
            pltpu.VMEM((SLOTS, ROW_CHUNK, HULL_W), jnp.float32),
            pltpu.VMEM((16,), jnp.float32),
            pltpu.SemaphoreType.DMA((SLOTS,)),
            pltpu.SemaphoreType.DMA((SLOTS,)),
        ],
    )(_sc_body)
    return sc_kernel(x, alpha, beta)


def _merge_kernel(c_ref, x_ref, o_ref):
    del x_ref
    o_ref[...] = c_ref[...]


MERGE_ROWS = 1024
MERGE_COLS = 128


def kernel(x, alpha, beta):
    m, n = x.shape
    c = _sc_band_hull(x, alpha, beta)
    first_block = HULL_START // MERGE_COLS
    return pl.pallas_call(
        _merge_kernel,
        grid=(m // MERGE_ROWS, HULL_W // MERGE_COLS),
        in_specs=[
            pl.BlockSpec((MERGE_ROWS, MERGE_COLS), lambda i, j: (i, j)),
            pl.BlockSpec(memory_space=pltpu.HBM),
        ],
        out_specs=pl.BlockSpec(
            (MERGE_ROWS, MERGE_COLS), lambda i, j: (i, first_block + j)),
        out_shape=jax.ShapeDtypeStruct((m, n), x.dtype),
        input_output_aliases={1: 0},
        compiler_params=pltpu.CompilerParams(
            dimension_semantics=("arbitrary", "arbitrary"),
        ),
    )(c, x)


# speedup vs baseline: 1.2193x; 1.0255x over previous
import functools

import jax
import jax.numpy as jnp
from jax import lax
from jax.experimental import pallas as pl
from jax.experimental.pallas import tpu as pltpu
from jax.experimental.pallas import tpu_sc as plsc

NUM_CLASSES = 10000
CLASSES_PER_TASK = 1000
CURRENT_TASK = 1
BAND_START = CURRENT_TASK * CLASSES_PER_TASK
BAND_END = BAND_START + CLASSES_PER_TASK

LANES = 16
HULL_START = (BAND_START // 128) * 128
HULL_END = -(-BAND_END // 128) * 128
HULL_W = HULL_END - HULL_START
LOC0 = ((BAND_START - HULL_START) // LANES) * LANES
N_FULL = (BAND_END - HULL_START - (LOC0 + LANES)) // LANES

ROWS = 4096
N_WORKERS = 32
ROWS_PER_WORKER = ROWS // N_WORKERS
ROW_CHUNK = 8
N_CHUNKS = ROWS_PER_WORKER // ROW_CHUNK
SLOTS = 4
PRIME = 2


def _sc_body(x_hbm, alpha_hbm, beta_hbm, c_hbm, buf, ab_v, in_sem, out_sem):
    wid = lax.axis_index("s") * 2 + lax.axis_index("c")
    base = wid * ROWS_PER_WORKER

    def in_dma(k, slot):
        return pltpu.make_async_copy(
            x_hbm.at[pl.ds(base + k * ROW_CHUNK, ROW_CHUNK),
                     pl.ds(HULL_START, HULL_W)],
            buf.at[slot],
            in_sem.at[slot],
        )

    def out_dma(k, slot):
        return pltpu.make_async_copy(
            buf.at[slot],
            c_hbm.at[pl.ds(base + k * ROW_CHUNK, ROW_CHUNK), :],
            out_sem.at[slot],
        )

    pltpu.sync_copy(alpha_hbm, ab_v.at[pl.ds(0, 1)])
    pltpu.sync_copy(beta_hbm, ab_v.at[pl.ds(8, 1)])
    ab = ab_v[...]
    a = ab[0]
    b = ab[8]
    edge_mask = lax.iota(jnp.int32, LANES) >= (BAND_START - HULL_START - LOC0)

    for d in range(PRIME):
        in_dma(d, d).start()

    def correct_rows(slot):
        for r in range(ROW_CHUNK):
            v = buf[slot, r, pl.ds(LOC0, LANES)]
            buf[slot, r, pl.ds(LOC0, LANES)] = jnp.where(
                edge_mask, v * a + b, v)
            for j in range(N_FULL):
                off = (LOC0 + LANES) + j * LANES
                buf[slot, r, pl.ds(off, LANES)] = (
                    buf[slot, r, pl.ds(off, LANES)] * a + b)

    def outer(k0, _):
        for d in range(SLOTS):
            k = k0 + d

            @pl.when(k + PRIME < N_CHUNKS)
            def _():
                @pl.when(k + PRIME >= SLOTS)
                def _():
                    out_dma(k + PRIME - SLOTS, (k + PRIME) % SLOTS).wait()

                in_dma(k + PRIME, (k + PRIME) % SLOTS).start()

            in_dma(k, d).wait()
            correct_rows(d)
            out_dma(k, d).start()
        return 0

    lax.fori_loop(0, N_CHUNKS // SLOTS, lambda i, c: outer(i * SLOTS, c), 0)

    for k in range(N_CHUNKS - SLOTS, N_CHUNKS):
        out_dma(k, k % SLOTS).wait()


def _sc_band_hull(x, alpha, beta):
    mesh = plsc.VectorSubcoreMesh(core_axis_name="c", subcore_axis_name="s")
    sc_kernel = functools.partial(
        pl.kernel,
        mesh=mesh,
        out_type=jax.ShapeDtypeStruct((ROWS, HULL_W), jnp.float32),
        scratch_types=[
            pltpu.VMEM((SLOTS, ROW_CHUNK, HULL_W), jnp.float32),
            pltpu.VMEM((16,), jnp.float32),
            pltpu.SemaphoreType.DMA((SLOTS,)),
            pltpu.SemaphoreType.DMA((SLOTS,)),
        ],
    )(_sc_body)
    return sc_kernel(x, alpha, beta)


def _merge_kernel(c_ref, x_ref, o_ref):
    del x_ref
    o_ref[...] = c_ref[...]


MERGE_ROWS = 2048
MERGE_COLS = 128


def kernel(x, alpha, beta):
    m, n = x.shape
    c = _sc_band_hull(x, alpha, beta)
    y = jnp.copy(x)
    first_block = HULL_START // MERGE_COLS
    return pl.pallas_call(
        _merge_kernel,
        grid=(m // MERGE_ROWS, HULL_W // MERGE_COLS),
        in_specs=[
            pl.BlockSpec((MERGE_ROWS, MERGE_COLS), lambda i, j: (i, j)),
            pl.BlockSpec(memory_space=pltpu.HBM),
        ],
        out_specs=pl.BlockSpec(
            (MERGE_ROWS, MERGE_COLS), lambda i, j: (i, first_block + j)),
        out_shape=jax.ShapeDtypeStruct((m, n), x.dtype),
        input_output_aliases={1: 0},
        compiler_params=pltpu.CompilerParams(
            dimension_semantics=("arbitrary", "arbitrary"),
        ),
    )(c, y)
